# baseline (speedup 1.0000x reference)
import functools

import jax
import jax.numpy as jnp
import numpy as np
from jax import lax
from jax.experimental import pallas as pl
from jax.experimental.pallas import tpu as pltpu
from jax.experimental.pallas import tpu_sc as plsc

N, E, D, ED, H = 10000, 320000, 128, 16, 256

NTILES = 32
EPT = E // NTILES
CHUNK = 40
NREAL = EPT // CHUNK
NPROC = 252
NPAIR = NPROC // 2
RING = 3
IRING = 6
NPAD = 10240
ROWS_PT = NPAD // (NTILES // 2)
LANES = 16


def _edge_proj(attr_even, attr_odd, We1, be1, We2, be2):
  BE = 4000

  def body(ae_ref, ao_ref, w1_ref, b1_ref, w2_ref, b2_ref, o1_ref, o2_ref):
    ae = ae_ref[...]
    ao = ao_ref[...]

    def rnd(u):
      return (u + jnp.uint32(0x7FFF) + ((u >> 16) & jnp.uint32(1))) >> 16

    def packed(w, b):
      ea = jnp.dot(ae, w[...], preferred_element_type=jnp.float32) + b[...]
      eo = jnp.dot(ao, w[...], preferred_element_type=jnp.float32) + b[...]
      ua = rnd(lax.bitcast_convert_type(ea, jnp.uint32))
      uo = rnd(lax.bitcast_convert_type(eo, jnp.uint32))
      return lax.bitcast_convert_type(ua | (uo << 16), jnp.int32)

    o1_ref[...] = packed(w1_ref, b1_ref)
    o2_ref[...] = packed(w2_ref, b2_ref)

  return pl.pallas_call(
      body,
      grid=(E // 2 // BE,),
      in_specs=[
          pl.BlockSpec((BE, ED), lambda i: (i, 0)),
          pl.BlockSpec((BE, ED), lambda i: (i, 0)),
          pl.BlockSpec((ED, D), lambda i: (0, 0)),
          pl.BlockSpec((1, D), lambda i: (0, 0)),
          pl.BlockSpec((ED, D), lambda i: (0, 0)),
          pl.BlockSpec((1, D), lambda i: (0, 0)),
      ],
      out_specs=[
          pl.BlockSpec((BE, D), lambda i: (i, 0)),
          pl.BlockSpec((BE, D), lambda i: (i, 0)),
      ],
      out_shape=[jax.ShapeDtypeStruct((E // 2, D), jnp.int32)] * 2,
  )(attr_even, attr_odd, We1, be1.reshape(1, D), We2, be2.reshape(1, D))


def _mlp(x, parts, Wa, ba, Wb, bb):
  BN = 2000

  def body(x_ref, p_ref, wa_ref, ba_ref, wb_ref, bb_ref, o_ref):
    hv = x_ref[...] + p_ref[0] + p_ref[1]
    t = jnp.maximum(
        jnp.dot(hv, wa_ref[...], preferred_element_type=jnp.float32)
        + ba_ref[...], 0.0)
    o_ref[...] = jnp.maximum(
        jnp.dot(t, wb_ref[...], preferred_element_type=jnp.float32)
        + bb_ref[...], 0.0)

  return pl.pallas_call(
      body,
      grid=(N // BN,),
      in_specs=[
          pl.BlockSpec((BN, D), lambda i: (i, 0)),
          pl.BlockSpec((2, BN, D), lambda i: (0, i, 0)),
          pl.BlockSpec((D, H), lambda i: (0, 0)),
          pl.BlockSpec((1, H), lambda i: (0, 0)),
          pl.BlockSpec((H, D), lambda i: (0, 0)),
          pl.BlockSpec((1, D), lambda i: (0, 0)),
      ],
      out_specs=pl.BlockSpec((BN, D), lambda i: (i, 0)),
      out_shape=jax.ShapeDtypeStruct((N, D), jnp.float32),
  )(x, parts, Wa, ba.reshape(1, H), Wb, bb.reshape(1, D))


def _sc_aggregate(x, ei, idx4d):
  mesh = plsc.VectorSubcoreMesh(core_axis_name="c", subcore_axis_name="s")

  @functools.partial(
      pl.kernel,
      mesh=mesh,
      out_type=jax.ShapeDtypeStruct((2, NPAD, D), jnp.float32),
      scratch_types=[
          pltpu.VMEM((IRING, 2, CHUNK), jnp.int32),
          pltpu.VMEM((RING, CHUNK, D), jnp.float32),
          pltpu.VMEM((RING, CHUNK, D), jnp.int32),
          pltpu.VMEM_SHARED((NPAD, D), jnp.float32),
          pltpu.SemaphoreType.DMA((IRING,)),
          pltpu.SemaphoreType.DMA((RING,)),
          pltpu.SemaphoreType.DMA((RING,)),
          pltpu.SemaphoreType.DMA((RING,)),
      ],
  )
  def k(x_hbm, e_hbm, idx_hbm, out_hbm,
        idxb, xg, ebuf, agg, isem, gsem, esem, ssem):
    c = lax.axis_index("c")
    s = lax.axis_index("s")
    wid = s * 2 + c
    edge0 = wid * EPT

    def fire_idx(kk, islot):
      return pltpu.async_copy(idx_hbm.at[wid, kk], idxb.at[islot],
                              isem.at[islot])

    def wait_idx(kk, islot):
      pltpu.make_async_copy(idx_hbm.at[wid, kk], idxb.at[islot],
                            isem.at[islot]).wait()

    def fire_g(kk, gslot, islot):
      pltpu.async_copy(x_hbm.at[idxb.at[islot, 0]], xg.at[gslot],
                       gsem.at[gslot])

    def wait_g(kk, gslot, islot):
      pltpu.make_async_copy(x_hbm.at[idxb.at[islot, 0]], xg.at[gslot],
                            gsem.at[gslot]).wait()

    erow0 = wid * (EPT // 2)

    def fire_e(t, eslot):
      eoff = erow0 + jnp.minimum(t, NPAIR - 2) * CHUNK
      pltpu.async_copy(e_hbm.at[pl.ds(eoff, CHUNK), :], ebuf.at[eslot],
                       esem.at[eslot])

    def wait_e(t, eslot):
      eoff = erow0 + jnp.minimum(t, NPAIR - 2) * CHUNK
      pltpu.make_async_copy(e_hbm.at[pl.ds(eoff, CHUNK), :], ebuf.at[eslot],
                            esem.at[eslot]).wait()

    def fire_scatter(gslot, islot):
      pltpu.async_copy(xg.at[gslot], agg.at[idxb.at[islot, 1]],
                       ssem.at[gslot], add=True)

    def wait_scatter(gslot, islot):
      pltpu.make_async_copy(xg.at[gslot], agg.at[idxb.at[islot, 1]],
                            ssem.at[gslot]).wait()

    zero16 = jnp.zeros((LANES,), jnp.float32)

    def zrow(r, carry):
      for cc in range(D // LANES):
        xg[0, r, pl.ds(cc * LANES, LANES)] = zero16
      return carry

    lax.fori_loop(0, CHUNK, zrow, 0)
    row0 = s * ROWS_PT
    for z in range(ROWS_PT // CHUNK):
      pltpu.sync_copy(xg.at[0], agg.at[pl.ds(row0 + z * CHUNK, CHUNK), :])
    plsc.subcore_barrier()

    for j in range(4):
      fire_idx(j, j)
    fire_e(0, 0)
    fire_e(1, 1)
    for j in range(2):
      wait_idx(j, j)
      fire_g(j, j, j)

    def iter_body(m, carry):
      for phase in range(IRING):
        kk = m * IRING + phase
        gslot = phase % RING
        wait_g(kk, gslot, phase)
        if phase % 2 == 0:
          wait_e(kk // 2, (phase // 2) % RING)
        eslot = (phase // 2) % RING
        ebase = (phase % 2) * (CHUNK // 2)

        def rowbody(q, rcarry):
          for g in range(D // LANES):
            sl = pl.ds(g * LANES, LANES)
            we = ebuf[eslot, ebase + q, sl]
            e0 = lax.bitcast_convert_type(we << 16, jnp.float32)
            e1 = lax.bitcast_convert_type(we & jnp.int32(-65536),
                                          jnp.float32)
            xg[gslot, 2 * q, sl] = jnp.maximum(xg[gslot, 2 * q, sl] + e0,
                                               0.0)
            xg[gslot, 2 * q + 1, sl] = jnp.maximum(
                xg[gslot, 2 * q + 1, sl] + e1, 0.0)
          return rcarry

        lax.fori_loop(0, CHUNK // 2, rowbody, 0)
        fire_scatter(gslot, phase)

        @pl.when(kk >= 1)
        def _():
          wait_scatter((phase - 1) % RING, (phase - 1) % IRING)

        @pl.when(kk <= NPROC - 3)
        def _():
          wait_idx(kk + 2, (phase + 2) % IRING)
          fire_g(kk + 2, (phase + 2) % RING, (phase + 2) % IRING)

        if phase % 2 == 1:
          @pl.when(kk <= NPROC - 5)
          def _():
            fire_e(kk // 2 + 2, (phase // 2 + 2) % RING)

        @pl.when(kk <= NPROC - 5)
        def _():
          fire_idx(kk + 4, (phase + 4) % IRING)
      return carry

    lax.fori_loop(0, NPROC // IRING, iter_body, 0)
    wait_scatter((NPROC - 1) % RING, (NPROC - 1) % IRING)
    plsc.subcore_barrier()

    pltpu.sync_copy(agg.at[pl.ds(row0, ROWS_PT), :],
                    out_hbm.at[c, pl.ds(row0, ROWS_PT), :])

  return k(x, ei, idx4d)


def kernel(node_feats, edge_index, edge_attr, We1, be1, W1a, b1a, W1b, b1b,
           We2, be2, W2a, b2a, W2b, b2b):
  edge_attr = edge_attr.reshape(E, ED)
  npe = (NPROC - NREAL) * CHUNK
  src = edge_index[0].reshape(NTILES, EPT)
  dst = edge_index[1].reshape(NTILES, EPT)
  pad_src = jnp.zeros((NTILES, npe), jnp.int32)
  pad_dst = jnp.broadcast_to(
      N + (jnp.arange(npe, dtype=jnp.int32) % (NPAD - N)), (NTILES, npe))
  srcp = jnp.concatenate([src, pad_src], 1).reshape(NTILES, NPROC, 1, CHUNK)
  dstp = jnp.concatenate([dst, pad_dst], 1).reshape(NTILES, NPROC, 1, CHUNK)
  idx4d = jnp.concatenate([srcp, dstp], 2)

  attr2 = edge_attr.reshape(E // 2, 2, ED)
  e1i, e2i = _edge_proj(attr2[:, 0], attr2[:, 1], We1, be1, We2, be2)

  p1 = _sc_aggregate(node_feats, e1i, idx4d)
  h1 = _mlp(node_feats, p1, W1a, b1a, W1b, b1b)

  p2 = _sc_aggregate(h1, e2i, idx4d)
  h2 = _mlp(h1, p2, W2a, b2a, W2b, b2b)
  return h2


# speedup vs baseline: 1.2406x; 1.2406x over previous
import functools

import jax
import jax.numpy as jnp
from jax import lax
from jax.experimental import pallas as pl
from jax.experimental.pallas import tpu as pltpu
from jax.experimental.pallas import tpu_sc as plsc

N, E, D, ED, H = 10000, 320000, 128, 16, 256

NTILES = 32
EPT = E // NTILES
CHUNK = 40
NREAL = EPT // CHUNK
NPROC = 252
RING = 3
IRING = 6
NPAD = 10240
ROWS_PT = NPAD // (NTILES // 2)
LANES = 16


def _edge_proj(edge_attr, We1, be1, We2, be2):
  BE = 8000

  def body(a_ref, w1_ref, b1_ref, w2_ref, b2_ref, o1_ref, o2_ref):
    a = a_ref[...]
    o1_ref[...] = jnp.dot(a, w1_ref[...],
                          preferred_element_type=jnp.float32) + b1_ref[...]
    o2_ref[...] = jnp.dot(a, w2_ref[...],
                          preferred_element_type=jnp.float32) + b2_ref[...]

  return pl.pallas_call(
      body,
      grid=(E // BE,),
      in_specs=[
          pl.BlockSpec((BE, ED), lambda i: (i, 0)),
          pl.BlockSpec((ED, D), lambda i: (0, 0)),
          pl.BlockSpec((1, D), lambda i: (0, 0)),
          pl.BlockSpec((ED, D), lambda i: (0, 0)),
          pl.BlockSpec((1, D), lambda i: (0, 0)),
      ],
      out_specs=[
          pl.BlockSpec((BE, D), lambda i: (i, 0)),
          pl.BlockSpec((BE, D), lambda i: (i, 0)),
      ],
      out_shape=[jax.ShapeDtypeStruct((E, D), jnp.float32)] * 2,
  )(edge_attr, We1, be1.reshape(1, D), We2, be2.reshape(1, D))


def _mlp(x, parts, Wa, ba, Wb, bb):
  BN = 2000

  def body(x_ref, p_ref, wa_ref, ba_ref, wb_ref, bb_ref, o_ref):
    hv = x_ref[...] + p_ref[0] + p_ref[1]
    t = jnp.maximum(
        jnp.dot(hv, wa_ref[...], preferred_element_type=jnp.float32)
        + ba_ref[...], 0.0)
    o_ref[...] = jnp.maximum(
        jnp.dot(t, wb_ref[...], preferred_element_type=jnp.float32)
        + bb_ref[...], 0.0)

  return pl.pallas_call(
      body,
      grid=(N // BN,),
      in_specs=[
          pl.BlockSpec((BN, D), lambda i: (i, 0)),
          pl.BlockSpec((2, BN, D), lambda i: (0, i, 0)),
          pl.BlockSpec((D, H), lambda i: (0, 0)),
          pl.BlockSpec((1, H), lambda i: (0, 0)),
          pl.BlockSpec((H, D), lambda i: (0, 0)),
          pl.BlockSpec((1, D), lambda i: (0, 0)),
      ],
      out_specs=pl.BlockSpec((BN, D), lambda i: (i, 0)),
      out_shape=jax.ShapeDtypeStruct((N, D), jnp.float32),
  )(x, parts, Wa, ba.reshape(1, H), Wb, bb.reshape(1, D))


def _sc_aggregate(x, e, idx4d):
  mesh = plsc.VectorSubcoreMesh(core_axis_name="c", subcore_axis_name="s")

  @functools.partial(
      pl.kernel,
      mesh=mesh,
      out_type=jax.ShapeDtypeStruct((2, NPAD, D), jnp.float32),
      scratch_types=[
          pltpu.VMEM((IRING, 2, CHUNK), jnp.int32),
          pltpu.VMEM((RING, CHUNK, D), jnp.float32),
          pltpu.VMEM((RING, CHUNK, D), jnp.float32),
          pltpu.VMEM_SHARED((NPAD, D), jnp.float32),
          pltpu.SemaphoreType.DMA((IRING,)),
          pltpu.SemaphoreType.DMA((RING,)),
          pltpu.SemaphoreType.DMA((RING,)),
          pltpu.SemaphoreType.DMA((RING,)),
      ],
  )
  def k(x_hbm, e_hbm, idx_hbm, out_hbm,
        idxb, xg, ebuf, agg, isem, gsem, esem, ssem):
    c = lax.axis_index("c")
    s = lax.axis_index("s")
    wid = s * 2 + c
    edge0 = wid * EPT

    def fire_idx(kk, islot):
      return pltpu.async_copy(idx_hbm.at[wid, kk], idxb.at[islot],
                              isem.at[islot])

    def wait_idx(kk, islot):
      pltpu.make_async_copy(idx_hbm.at[wid, kk], idxb.at[islot],
                            isem.at[islot]).wait()

    def fire_ge(kk, gslot, islot):
      eoff = edge0 + jnp.minimum(kk, NREAL - 1) * CHUNK
      pltpu.async_copy(e_hbm.at[pl.ds(eoff, CHUNK), :], ebuf.at[gslot],
                       esem.at[gslot])
      pltpu.async_copy(x_hbm.at[idxb.at[islot, 0]], xg.at[gslot],
                       gsem.at[gslot])

    def wait_ge(kk, gslot, islot):
      eoff = edge0 + jnp.minimum(kk, NREAL - 1) * CHUNK
      pltpu.make_async_copy(e_hbm.at[pl.ds(eoff, CHUNK), :], ebuf.at[gslot],
                            esem.at[gslot]).wait()
      pltpu.make_async_copy(x_hbm.at[idxb.at[islot, 0]], xg.at[gslot],
                            gsem.at[gslot]).wait()

    def fire_scatter(gslot, islot):
      pltpu.async_copy(xg.at[gslot], agg.at[idxb.at[islot, 1]],
                       ssem.at[gslot], add=True)

    def wait_scatter(gslot, islot):
      pltpu.make_async_copy(xg.at[gslot], agg.at[idxb.at[islot, 1]],
                            ssem.at[gslot]).wait()

    zero16 = jnp.zeros((LANES,), jnp.float32)

    def zrow(r, carry):
      for cc in range(D // LANES):
        xg[0, r, pl.ds(cc * LANES, LANES)] = zero16
      return carry

    lax.fori_loop(0, CHUNK, zrow, 0)
    row0 = s * ROWS_PT
    for z in range(ROWS_PT // CHUNK):
      pltpu.sync_copy(xg.at[0], agg.at[pl.ds(row0 + z * CHUNK, CHUNK), :])
    plsc.subcore_barrier()

    for j in range(4):
      fire_idx(j, j)
    for j in range(2):
      wait_idx(j, j)
      fire_ge(j, j, j)

    def iter_body(m, carry):
      for phase in range(IRING):
        kk = m * IRING + phase
        gslot = phase % RING
        wait_ge(kk, gslot, phase)

        def rowbody(r, rcarry):
          for cc in range(D // LANES):
            sl = pl.ds(cc * LANES, LANES)
            xg[gslot, r, sl] = jnp.maximum(
                xg[gslot, r, sl] + ebuf[gslot, r, sl], 0.0)
          return rcarry

        lax.fori_loop(0, CHUNK, rowbody, 0)
        fire_scatter(gslot, phase)

        @pl.when(kk >= 1)
        def _():
          wait_scatter((phase - 1) % RING, (phase - 1) % IRING)

        @pl.when(kk <= NPROC - 3)
        def _():
          wait_idx(kk + 2, (phase + 2) % IRING)
          fire_ge(kk + 2, (phase + 2) % RING, (phase + 2) % IRING)

        @pl.when(kk <= NPROC - 5)
        def _():
          fire_idx(kk + 4, (phase + 4) % IRING)
      return carry

    lax.fori_loop(0, NPROC // IRING, iter_body, 0)
    wait_scatter((NPROC - 1) % RING, (NPROC - 1) % IRING)
    plsc.subcore_barrier()

    pltpu.sync_copy(agg.at[pl.ds(row0, ROWS_PT), :],
                    out_hbm.at[c, pl.ds(row0, ROWS_PT), :])

  return k(x, e, idx4d)


def kernel(node_feats, edge_index, edge_attr, We1, be1, W1a, b1a, W1b, b1b,
           We2, be2, W2a, b2a, W2b, b2b):
  edge_attr = edge_attr.reshape(E, ED)
  npe = (NPROC - NREAL) * CHUNK
  src = edge_index[0].reshape(NTILES, EPT)
  dst = edge_index[1].reshape(NTILES, EPT)
  pad_src = jnp.zeros((NTILES, npe), jnp.int32)
  pad_dst = jnp.broadcast_to(
      N + (jnp.arange(npe, dtype=jnp.int32) % (NPAD - N)), (NTILES, npe))
  srcp = jnp.concatenate([src, pad_src], 1).reshape(NTILES, NPROC, 1, CHUNK)
  dstp = jnp.concatenate([dst, pad_dst], 1).reshape(NTILES, NPROC, 1, CHUNK)
  idx4d = jnp.concatenate([srcp, dstp], 2)

  e1, e2 = _edge_proj(edge_attr, We1, be1, We2, be2)

  p1 = _sc_aggregate(node_feats, e1, idx4d)
  h1 = _mlp(node_feats, p1, W1a, b1a, W1b, b1b)

  p2 = _sc_aggregate(h1, e2, idx4d)
  h2 = _mlp(h1, p2, W2a, b2a, W2b, b2b)
  return h2
